# baseline (speedup 1.0000x reference)
import jax
import jax.numpy as jnp
from jax.experimental import pallas as pl

_B, _H, _W, _C = 32, 512, 512, 3
_BLK_H = 128


def _copy_kernel(x_ref, o_ref):
    o_ref[...] = x_ref[...]


def kernel(inputs):
    x = jnp.transpose(inputs, (0, 3, 1, 2))
    out = pl.pallas_call(
        _copy_kernel,
        grid=(_B, _C, _H // _BLK_H),
        in_specs=[
            pl.BlockSpec((None, None, _BLK_H, _W), lambda b, c, h: (b, 2 - c, h, 0))
        ],
        out_specs=pl.BlockSpec((None, None, _BLK_H, _W), lambda b, c, h: (b, c, h, 0)),
        out_shape=jax.ShapeDtypeStruct((_B, _C, _H, _W), jnp.float32),
    )(x)
    return jnp.transpose(out, (0, 2, 3, 1))


# speedup vs baseline: 2.6814x; 2.6814x over previous
import functools

import jax
import jax.numpy as jnp
from jax import lax
from jax.experimental import pallas as pl
from jax.experimental.pallas import tpu as pltpu
from jax.experimental.pallas import tpu_sc as plsc

_B, _H, _W, _C = 32, 512, 512, 3
_BLOCKS = [(0, 120), (120, 120), (240, 120), (360, 120), (480, 32)]


def _sc_body(in_hbm, out_hbm, buf, insem, outsem0, outsem1):
    cid = lax.axis_index("c")
    sid = lax.axis_index("s")
    wid = sid * 2 + cid

    plan = [(c, h0, rows) for c in range(_C) for (h0, rows) in _BLOCKS]
    n = len(plan)

    def src(i):
        c, h0, rows = plan[i]
        return in_hbm.at[wid, 2 - c, pl.ds(h0, rows)]

    def dst(i):
        c, h0, rows = plan[i]
        return out_hbm.at[wid, c, pl.ds(h0, rows)]

    def slot(i):
        return buf.at[i % 2, : plan[i][2]]

    outsems = (outsem0, outsem1)
    reads = [None] * n
    writes = [None, None]
    reads[0] = pltpu.async_copy(src(0), slot(0), insem)
    reads[1] = pltpu.async_copy(src(1), slot(1), insem)
    for i in range(n):
        s = i % 2
        reads[i].wait()
        writes[s] = pltpu.async_copy(slot(i), dst(i), outsems[s])
        if i + 2 < n:
            s2 = (i + 2) % 2
            writes[s2].wait()
            writes[s2] = None
            reads[i + 2] = pltpu.async_copy(src(i + 2), slot(i + 2), insem)
    for cp in writes:
        if cp is not None:
            cp.wait()


def kernel(inputs):
    x = jnp.transpose(inputs, (0, 3, 1, 2))
    mesh = plsc.VectorSubcoreMesh(core_axis_name="c", subcore_axis_name="s")
    run = functools.partial(
        pl.kernel,
        mesh=mesh,
        out_type=jax.ShapeDtypeStruct((_B, _C, _H, _W), jnp.float32),
        scratch_types=[
            pltpu.VMEM((2, 120, _W), jnp.float32),
            pltpu.SemaphoreType.DMA,
            pltpu.SemaphoreType.DMA,
            pltpu.SemaphoreType.DMA,
        ],
        compiler_params=pltpu.CompilerParams(
            needs_layout_passes=False, use_tc_tiling_on_sc=True
        ),
    )(_sc_body)
    out = run(x)
    return jnp.transpose(out, (0, 2, 3, 1))


# speedup vs baseline: 2.6845x; 1.0011x over previous
import functools

import jax
import jax.numpy as jnp
from jax import lax
from jax.experimental import pallas as pl
from jax.experimental.pallas import tpu as pltpu
from jax.experimental.pallas import tpu_sc as plsc

_B, _H, _W, _C = 32, 512, 512, 3
_BLK_H = 64
_NBLK = _H // _BLK_H


def _sc_body(in_hbm, out_hbm, buf, insem, outsem0, outsem1, outsem2):
    cid = lax.axis_index("c")
    sid = lax.axis_index("s")
    wid = sid * 2 + cid

    n = _C * _NBLK

    def src(i):
        c, hb = divmod(i, _NBLK)
        return in_hbm.at[wid, 2 - c, pl.ds(hb * _BLK_H, _BLK_H)]

    def dst(i):
        c, hb = divmod(i, _NBLK)
        return out_hbm.at[wid, c, pl.ds(hb * _BLK_H, _BLK_H)]

    outsems = (outsem0, outsem1, outsem2)
    reads = [None] * n
    writes = [None, None, None]
    reads[0] = pltpu.async_copy(src(0), buf.at[0], insem)
    reads[1] = pltpu.async_copy(src(1), buf.at[1], insem)
    for i in range(n):
        s = i % 3
        reads[i].wait()
        writes[s] = pltpu.async_copy(buf.at[s], dst(i), outsems[s])
        if i + 2 < n:
            s2 = (i + 2) % 3
            if writes[s2] is not None:
                writes[s2].wait()
                writes[s2] = None
            reads[i + 2] = pltpu.async_copy(src(i + 2), buf.at[s2], insem)
    for cp in writes:
        if cp is not None:
            cp.wait()


def kernel(inputs):
    x = jnp.transpose(inputs, (0, 3, 1, 2))
    mesh = plsc.VectorSubcoreMesh(core_axis_name="c", subcore_axis_name="s")
    run = functools.partial(
        pl.kernel,
        mesh=mesh,
        out_type=jax.ShapeDtypeStruct((_B, _C, _H, _W), jnp.float32),
        scratch_types=[
            pltpu.VMEM((3, _BLK_H, _W), jnp.float32),
            pltpu.SemaphoreType.DMA,
            pltpu.SemaphoreType.DMA,
            pltpu.SemaphoreType.DMA,
            pltpu.SemaphoreType.DMA,
        ],
        compiler_params=pltpu.CompilerParams(
            needs_layout_passes=False, use_tc_tiling_on_sc=True
        ),
    )(_sc_body)
    out = run(x)
    return jnp.transpose(out, (0, 2, 3, 1))
